# baseline (speedup 1.0000x reference)
import functools

import jax
import jax.numpy as jnp
from jax import lax
from jax.experimental import pallas as pl
from jax.experimental.pallas import tpu as pltpu
from jax.experimental.pallas import tpu_sc as plsc

NUM_ENTRIES = 100000
LEARNABLE_SIZE = 4
D = 768
BATCH = 4096
ROW = LEARNABLE_SIZE * D

NC = 2
NS = 16
NW = NC * NS
B_PER_W = BATCH // NW
CHUNK = 16
NCHUNK = B_PER_W // CHUNK

_mesh = plsc.VectorSubcoreMesh(core_axis_name="c", subcore_axis_name="s")


@functools.partial(
    pl.kernel,
    mesh=_mesh,
    out_type=jax.ShapeDtypeStruct((BATCH, ROW), jnp.float32),
    scratch_types=[
        pltpu.VMEM((NCHUNK, CHUNK), jnp.int32),
        pltpu.VMEM((CHUNK, ROW), jnp.float32),
        pltpu.SemaphoreType.DMA,
    ],
)
def _gather_kernel(idx_hbm, table_hbm, out_hbm, idx_v, rows_v, gsem):
    wid = lax.axis_index("s") * NC + lax.axis_index("c")
    pltpu.sync_copy(idx_hbm.at[wid], idx_v)
    base = wid * B_PER_W
    for c in range(NCHUNK):
        pltpu.async_copy(table_hbm.at[idx_v.at[c]], rows_v, gsem).wait()
        pltpu.sync_copy(rows_v, out_hbm.at[pl.ds(base + c * CHUNK, CHUNK)])


def kernel(x, learnable_matrix):
    idx = x.astype(jnp.int32).reshape(NW, NCHUNK, CHUNK)
    table = learnable_matrix.reshape(NUM_ENTRIES, ROW)
    out = _gather_kernel(idx, table)
    return out.reshape(BATCH, LEARNABLE_SIZE, D)


# speedup vs baseline: 1.0101x; 1.0101x over previous
import functools

import jax
import jax.numpy as jnp
from jax import lax
from jax.experimental import pallas as pl
from jax.experimental.pallas import tpu as pltpu
from jax.experimental.pallas import tpu_sc as plsc

NUM_ENTRIES = 100000
LEARNABLE_SIZE = 4
D = 768
BATCH = 4096
ROW = LEARNABLE_SIZE * D

NC = 2
NS = 16
NW = NC * NS
B_PER_W = BATCH // NW
CHUNK = 8
NCHUNK = B_PER_W // CHUNK
NBUF = 4

_mesh = plsc.VectorSubcoreMesh(core_axis_name="c", subcore_axis_name="s")


@functools.partial(
    pl.kernel,
    mesh=_mesh,
    out_type=jax.ShapeDtypeStruct((BATCH, ROW), jnp.float32),
    scratch_types=[
        pltpu.VMEM((NCHUNK, CHUNK), jnp.int32),
        pltpu.VMEM((NBUF, CHUNK, ROW), jnp.float32),
    ]
    + [pltpu.SemaphoreType.DMA] * (2 * NBUF),
)
def _gather_kernel(idx_hbm, table_hbm, out_hbm, idx_v, rows_v, *sems):
    gsems = sems[:NBUF]
    osems = sems[NBUF:]
    wid = lax.axis_index("s") * NC + lax.axis_index("c")
    pltpu.sync_copy(idx_hbm.at[wid], idx_v)
    base = wid * B_PER_W

    def gather(c, b):
        return pltpu.async_copy(table_hbm.at[idx_v.at[c]], rows_v.at[b],
                                gsems[b])

    def out_copy(c, b):
        return pltpu.make_async_copy(
            rows_v.at[b], out_hbm.at[pl.ds(base + c * CHUNK, CHUNK)],
            osems[b])

    for b in range(NBUF):
        gather(b, b)
    for c in range(NCHUNK):
        b = c % NBUF
        pltpu.make_async_copy(table_hbm.at[idx_v.at[c]], rows_v.at[b],
                              gsems[b]).wait()
        out_copy(c, b).start()
        m = c + NBUF
        if m < NCHUNK:
            out_copy(c, b).wait()
            gather(m, b)
    for c in range(NCHUNK - NBUF, NCHUNK):
        out_copy(c, c % NBUF).wait()


def kernel(x, learnable_matrix):
    idx = x.astype(jnp.int32).reshape(NW, NCHUNK, CHUNK)
    table = learnable_matrix.reshape(NUM_ENTRIES, ROW)
    out = _gather_kernel(idx, table)
    return out.reshape(BATCH, LEARNABLE_SIZE, D)


# speedup vs baseline: 17.7291x; 17.5518x over previous
import functools

import jax
import jax.numpy as jnp
from jax import lax
from jax.experimental import pallas as pl
from jax.experimental.pallas import tpu as pltpu
from jax.experimental.pallas import tpu_sc as plsc

NUM_ENTRIES = 100000
LEARNABLE_SIZE = 4
D = 768
BATCH = 4096
ROW = LEARNABLE_SIZE * D

NC = 2
NS = 16
NW = NC * NS
B_PER_W = BATCH // NW
CHUNK = 8
NCHUNK = B_PER_W // CHUNK
NBUF = 4

_mesh = plsc.VectorSubcoreMesh(core_axis_name="c", subcore_axis_name="s")


@functools.partial(
    pl.kernel,
    mesh=_mesh,
    out_type=jax.ShapeDtypeStruct((BATCH, LEARNABLE_SIZE, D), jnp.float32),
    scratch_types=[
        pltpu.VMEM((NCHUNK, CHUNK), jnp.int32),
        pltpu.VMEM((NBUF, CHUNK, LEARNABLE_SIZE, D), jnp.float32),
    ]
    + [pltpu.SemaphoreType.DMA] * (2 * NBUF),
)
def _gather_kernel(idx_hbm, table_hbm, out_hbm, idx_v, rows_v, *sems):
    gsems = sems[:NBUF]
    osems = sems[NBUF:]
    wid = lax.axis_index("s") * NC + lax.axis_index("c")
    pltpu.sync_copy(idx_hbm.at[wid], idx_v)
    base = wid * B_PER_W

    def gather(c, b):
        return pltpu.async_copy(table_hbm.at[idx_v.at[c]], rows_v.at[b],
                                gsems[b])

    def out_copy(c, b):
        return pltpu.make_async_copy(
            rows_v.at[b], out_hbm.at[pl.ds(base + c * CHUNK, CHUNK)],
            osems[b])

    for b in range(NBUF):
        gather(b, b)
    for c in range(NCHUNK):
        b = c % NBUF
        pltpu.make_async_copy(table_hbm.at[idx_v.at[c]], rows_v.at[b],
                              gsems[b]).wait()
        out_copy(c, b).start()
        m = c + NBUF
        if m < NCHUNK:
            out_copy(c, b).wait()
            gather(m, b)
    for c in range(NCHUNK - NBUF, NCHUNK):
        out_copy(c, c % NBUF).wait()


def kernel(x, learnable_matrix):
    idx = x.astype(jnp.int32).reshape(NW, NCHUNK, CHUNK)
    return _gather_kernel(idx, learnable_matrix)
